# baseline (speedup 1.0000x reference)
import dataclasses
import functools

import jax
import jax.numpy as jnp
from jax import lax
from jax.experimental import pallas as pl
from jax.experimental.pallas import tpu as pltpu
from jax.experimental.pallas import tpu_sc as plsc

N_NODES = 10000
N_EDGES = 160000
F_IN = 256
HID = 16

NC, NS, LANES = 2, 16, 16
NW = NC * NS
WIN = 128
N_PAD = 10240
DUMP = N_NODES
ROWS_PER_TILE = N_PAD // NS
ZCHUNK = 128
NZ = ROWS_PER_TILE // ZCHUNK
EPT = N_EDGES // NW
VFULL = EPT // LANES
TAIL = EPT - VFULL * LANES

_mesh = plsc.VectorSubcoreMesh(core_axis_name="c", subcore_axis_name="s")

_sc_params = pltpu.CompilerParams(
    needs_layout_passes=False, use_tc_tiling_on_sc=False)


def _deg_body(ei, out, hs, hd, si, di, sem):
    cid = lax.axis_index("c")
    sid = lax.axis_index("s")
    wid = cid * NS + sid
    zeros = jnp.zeros((LANES,), jnp.float32)

    csi = pltpu.async_copy(ei.at[0, pl.ds(wid * EPT, EPT)], si.at[pl.ds(0, EPT)], sem)
    cdi = pltpu.async_copy(ei.at[1, pl.ds(wid * EPT, EPT)], di.at[pl.ds(0, EPT)], sem)

    @pl.loop(0, N_PAD // LANES, step=8)
    def _(i):
        for k in range(8):
            hs[pl.ds((i + k) * LANES, LANES)] = zeros
            hd[pl.ds((i + k) * LANES, LANES)] = zeros

    csi.wait()
    cdi.wait()
    ones = jnp.ones((LANES,), jnp.float32)

    @pl.loop(0, VFULL - VFULL % 4, step=4)
    def _(v):
        for k in range(4):
            plsc.addupdate_scatter(hs, [si[pl.ds((v + k) * LANES, LANES)]], ones)
            plsc.addupdate_scatter(hd, [di[pl.ds((v + k) * LANES, LANES)]], ones)

    @pl.loop(VFULL - VFULL % 4, VFULL)
    def _(v):
        plsc.addupdate_scatter(hs, [si[pl.ds(v * LANES, LANES)]], ones)
        plsc.addupdate_scatter(hd, [di[pl.ds(v * LANES, LANES)]], ones)

    tmask = lax.iota(jnp.int32, LANES) < TAIL
    plsc.addupdate_scatter(hs, [si[pl.ds(VFULL * LANES, LANES)]], ones, mask=tmask)
    plsc.addupdate_scatter(hd, [di[pl.ds(VFULL * LANES, LANES)]], ones, mask=tmask)

    pltpu.sync_copy(hs, out.at[wid])
    pltpu.sync_copy(hd, out.at[NW + wid])


def _sc_degrees(ei):
    k = pl.kernel(
        _deg_body,
        out_type=jax.ShapeDtypeStruct((2 * NW, N_PAD), jnp.float32),
        mesh=_mesh,
        scratch_types=[
            pltpu.VMEM((N_PAD,), jnp.float32),
            pltpu.VMEM((N_PAD,), jnp.float32),
            pltpu.VMEM((VFULL * LANES + LANES,), jnp.int32),
            pltpu.VMEM((VFULL * LANES + LANES,), jnp.int32),
            pltpu.SemaphoreType.DMA,
        ],
        compiler_params=_sc_params,
    )
    return k(ei)


def _agg_body(h, ei, out, si, di, rows, zb, acc, isem, gsem):
    cid = lax.axis_index("c")
    sid = lax.axis_index("s")
    wid = cid * NS + sid
    zeros = jnp.zeros((LANES,), jnp.float32)

    csi = pltpu.async_copy(ei.at[0, pl.ds(wid * EPT, EPT)], si, isem)
    cdi = pltpu.async_copy(ei.at[1, pl.ds(wid * EPT, EPT)], di, isem)

    @pl.loop(0, ZCHUNK, step=8)
    def _(i):
        for k in range(8):
            zb[i + k, :] = zeros

    csi.wait()
    g = pltpu.async_copy(h.at[si], rows, gsem)

    @pl.loop(0, NZ)
    def _(kk):
        pltpu.sync_copy(zb, acc.at[pl.ds(sid * ROWS_PER_TILE + kk * ZCHUNK, ZCHUNK)])

    cdi.wait()
    plsc.subcore_barrier()
    g.wait()
    pltpu.sync_copy(rows, acc.at[di], add=True)

    plsc.subcore_barrier()
    pltpu.sync_copy(acc.at[pl.ds(sid * ROWS_PER_TILE, ROWS_PER_TILE)],
                    out.at[cid, pl.ds(sid * ROWS_PER_TILE, ROWS_PER_TILE)])


def _sc_agg(h, ei):
    k = pl.kernel(
        _agg_body,
        out_type=jax.ShapeDtypeStruct((NC, N_PAD, HID), jnp.float32),
        mesh=_mesh,
        scratch_types=[
            pltpu.VMEM((EPT,), jnp.int32),
            pltpu.VMEM((EPT,), jnp.int32),
            pltpu.VMEM((EPT, HID), jnp.float32),
            pltpu.VMEM((ZCHUNK, HID), jnp.float32),
            pltpu.VMEM_SHARED((N_PAD, HID), jnp.float32),
            pltpu.SemaphoreType.DMA,
            pltpu.SemaphoreType.DMA,
        ],
        compiler_params=_sc_params,
    )
    return k(h, ei)


_BM = 1024
_BMP = _BM // (128 // HID)
PROWS = N_PAD // (128 // HID)


def _packmask():
    r8 = 128 // HID
    m_row = lax.broadcasted_iota(jnp.int32, (_BM, 128), 0)
    c_col = lax.broadcasted_iota(jnp.int32, (_BM, 128), 1)
    return (m_row % r8 == c_col // HID).astype(jnp.float32)


def _grpsum(y):
    return jnp.sum(y.reshape(_BMP, 128 // HID, 128), axis=1)


def _mm0_body(x_ref, w1e_ref, mme_ref):
    mme_ref[...] = jnp.dot(x_ref[...], w1e_ref[...],
                           preferred_element_type=jnp.float32)


def _tc_mm0(features, W1e):
    grid = N_PAD // _BM
    return pl.pallas_call(
        _mm0_body,
        grid=(grid,),
        in_specs=[
            pl.BlockSpec((_BM, F_IN), lambda i: (i, 0)),
            pl.BlockSpec((F_IN, 128), lambda i: (0, 0)),
        ],
        out_specs=pl.BlockSpec((_BM, 128), lambda i: (i, 0)),
        out_shape=jax.ShapeDtypeStruct((N_PAD, 128), jnp.float32),
    )(features, W1e)


def _mm1_body(mme_ref, degp_ref, hp_ref, nrmd_ref, nrms_ref):
    degp = degp_ref[...]
    deg_s = jnp.maximum(jnp.sum(degp[:NW], axis=0), 1.0)
    deg_d = jnp.maximum(jnp.sum(degp[NW:], axis=0), 1.0)
    mask = _packmask()
    nrms_col = lax.rsqrt(deg_s)[:, None]
    nrmd_col = lax.rsqrt(deg_d)[:, None]
    nrms_ref[...] = _grpsum(nrms_col * mask)
    nrmd_ref[...] = _grpsum(nrmd_col * mask)
    hp_ref[...] = _grpsum(mme_ref[...] * (nrms_col * mask))


def _tc_mm1(mme, degp):
    grid = N_PAD // _BM
    return pl.pallas_call(
        _mm1_body,
        grid=(grid,),
        in_specs=[
            pl.BlockSpec((_BM, 128), lambda i: (i, 0)),
            pl.BlockSpec((2 * NW, _BM), lambda i: (0, i)),
        ],
        out_specs=[
            pl.BlockSpec((_BMP, 128), lambda i: (i, 0)),
            pl.BlockSpec((_BMP, 128), lambda i: (i, 0)),
            pl.BlockSpec((_BMP, 128), lambda i: (i, 0)),
        ],
        out_shape=[
            jax.ShapeDtypeStruct((PROWS, 128), jnp.float32),
            jax.ShapeDtypeStruct((PROWS, 128), jnp.float32),
            jax.ShapeDtypeStruct((PROWS, 128), jnp.float32),
        ],
    )(mme, degp)


def _mid_body(p_ref, nrmd_ref, nrms_ref, b1_ref, w2bd_ref, h2p_ref):
    agg = p_ref[0] + p_ref[1]
    t = jnp.maximum(agg * nrmd_ref[...] + b1_ref[...], 0.0)
    mm = jnp.dot(t, w2bd_ref[...], preferred_element_type=jnp.float32)
    h2p_ref[...] = mm * nrms_ref[...]


def _tc_mid(p1, nrmd_p, nrms_p, b1t, W2bd):
    grid = PROWS // _BMP
    return pl.pallas_call(
        _mid_body,
        grid=(grid,),
        in_specs=[
            pl.BlockSpec((NC, _BMP, 128), lambda i: (0, i, 0)),
            pl.BlockSpec((_BMP, 128), lambda i: (i, 0)),
            pl.BlockSpec((_BMP, 128), lambda i: (i, 0)),
            pl.BlockSpec((1, 128), lambda i: (0, 0)),
            pl.BlockSpec((128, 128), lambda i: (0, 0)),
        ],
        out_specs=pl.BlockSpec((_BMP, 128), lambda i: (i, 0)),
        out_shape=jax.ShapeDtypeStruct((PROWS, 128), jnp.float32),
    )(p1, nrmd_p, nrms_p, b1t, W2bd)


def _final_body(p_ref, nrmd_ref, b2_ref, out_ref):
    agg = p_ref[0] + p_ref[1]
    out_ref[...] = agg * nrmd_ref[...] + b2_ref[...]


def _tc_final(p2, nrmd_p, b2t):
    grid = PROWS // _BMP
    return pl.pallas_call(
        _final_body,
        grid=(grid,),
        in_specs=[
            pl.BlockSpec((NC, _BMP, 128), lambda i: (0, i, 0)),
            pl.BlockSpec((_BMP, 128), lambda i: (i, 0)),
            pl.BlockSpec((1, 128), lambda i: (0, 0)),
        ],
        out_specs=pl.BlockSpec((_BMP, 128), lambda i: (i, 0)),
        out_shape=jax.ShapeDtypeStruct((PROWS, 128), jnp.float32),
    )(p2, nrmd_p, b2t)


def kernel(features, edge_index, W1, b1, W2, b2):
    ei = edge_index.astype(jnp.int32)
    rep = 128 // HID
    b1t = jnp.tile(b1, rep).reshape(1, 128)
    b2t = jnp.tile(b2, rep).reshape(1, 128)
    W1e = jnp.tile(W1, (1, rep))
    W2bd = jnp.kron(jnp.eye(rep, dtype=jnp.float32), W2)

    degp = _sc_degrees(ei)
    mme = _tc_mm0(features, W1e)
    h1p, nrmd_p, nrms_p = _tc_mm1(mme, degp)
    p1 = _sc_agg(h1p.reshape(N_PAD, HID), ei)
    h2p = _tc_mid(p1.reshape(NC, PROWS, 128), nrmd_p, nrms_p, b1t, W2bd)
    p2 = _sc_agg(h2p.reshape(N_PAD, HID), ei)
    outp = _tc_final(p2.reshape(NC, PROWS, 128), nrmd_p, b2t)
    return outp.reshape(N_PAD, HID)[:N_NODES]


# speedup vs baseline: 1.0518x; 1.0518x over previous
import dataclasses
import functools

import jax
import jax.numpy as jnp
from jax import lax
from jax.experimental import pallas as pl
from jax.experimental.pallas import tpu as pltpu
from jax.experimental.pallas import tpu_sc as plsc

N_NODES = 10000
N_EDGES = 160000
F_IN = 256
HID = 16

NC, NS, LANES = 2, 16, 16
NW = NC * NS
WIN = 128
N_PAD = 10240
DUMP = N_NODES
ROWS_PER_TILE = N_PAD // NS
ZCHUNK = 128
NZ = ROWS_PER_TILE // ZCHUNK
EPT = N_EDGES // NW
VFULL = EPT // LANES
TAIL = EPT - VFULL * LANES

_mesh = plsc.VectorSubcoreMesh(core_axis_name="c", subcore_axis_name="s")

_sc_params = pltpu.CompilerParams(
    needs_layout_passes=False, use_tc_tiling_on_sc=False)


def _deg_body(ei, out, hs, hd, si, di, sem):
    cid = lax.axis_index("c")
    sid = lax.axis_index("s")
    wid = cid * NS + sid
    zeros = jnp.zeros((LANES,), jnp.float32)

    csi = pltpu.async_copy(ei.at[0, pl.ds(wid * EPT, EPT)], si.at[pl.ds(0, EPT)], sem)
    cdi = pltpu.async_copy(ei.at[1, pl.ds(wid * EPT, EPT)], di.at[pl.ds(0, EPT)], sem)

    @pl.loop(0, N_PAD // LANES, step=8)
    def _(i):
        for k in range(8):
            hs[pl.ds((i + k) * LANES, LANES)] = zeros
            hd[pl.ds((i + k) * LANES, LANES)] = zeros

    csi.wait()
    cdi.wait()
    ones = jnp.ones((LANES,), jnp.float32)

    @pl.loop(0, VFULL - VFULL % 4, step=4)
    def _(v):
        for k in range(4):
            plsc.addupdate_scatter(hs, [si[pl.ds((v + k) * LANES, LANES)]], ones)
            plsc.addupdate_scatter(hd, [di[pl.ds((v + k) * LANES, LANES)]], ones)

    @pl.loop(VFULL - VFULL % 4, VFULL)
    def _(v):
        plsc.addupdate_scatter(hs, [si[pl.ds(v * LANES, LANES)]], ones)
        plsc.addupdate_scatter(hd, [di[pl.ds(v * LANES, LANES)]], ones)

    tmask = lax.iota(jnp.int32, LANES) < TAIL
    plsc.addupdate_scatter(hs, [si[pl.ds(VFULL * LANES, LANES)]], ones, mask=tmask)
    plsc.addupdate_scatter(hd, [di[pl.ds(VFULL * LANES, LANES)]], ones, mask=tmask)

    pltpu.sync_copy(hs, out.at[wid])
    pltpu.sync_copy(hd, out.at[NW + wid])


def _sc_degrees(ei):
    k = pl.kernel(
        _deg_body,
        out_type=jax.ShapeDtypeStruct((2 * NW, N_PAD), jnp.float32),
        mesh=_mesh,
        scratch_types=[
            pltpu.VMEM((N_PAD,), jnp.float32),
            pltpu.VMEM((N_PAD,), jnp.float32),
            pltpu.VMEM((VFULL * LANES + LANES,), jnp.int32),
            pltpu.VMEM((VFULL * LANES + LANES,), jnp.int32),
            pltpu.SemaphoreType.DMA,
        ],
        compiler_params=_sc_params,
    )
    return k(ei)


def _agg_body(h, ei, out, si, di, rows, zb, acc, isem, gsem):
    cid = lax.axis_index("c")
    sid = lax.axis_index("s")
    wid = cid * NS + sid
    zeros = jnp.zeros((LANES,), jnp.float32)

    csi = pltpu.async_copy(ei.at[0, pl.ds(wid * EPT, EPT)], si, isem)
    cdi = pltpu.async_copy(ei.at[1, pl.ds(wid * EPT, EPT)], di, isem)

    @pl.loop(0, ZCHUNK, step=8)
    def _(i):
        for k in range(8):
            zb[i + k, :] = zeros

    csi.wait()
    g = pltpu.async_copy(h.at[si], rows, gsem)

    @pl.loop(0, NZ)
    def _(kk):
        pltpu.sync_copy(zb, acc.at[pl.ds(sid * ROWS_PER_TILE + kk * ZCHUNK, ZCHUNK)])

    cdi.wait()
    plsc.subcore_barrier()
    g.wait()
    pltpu.sync_copy(rows, acc.at[di], add=True)

    plsc.subcore_barrier()
    pltpu.sync_copy(acc.at[pl.ds(sid * ROWS_PER_TILE, ROWS_PER_TILE)],
                    out.at[cid, pl.ds(sid * ROWS_PER_TILE, ROWS_PER_TILE)])


def _sc_agg(h, ei):
    k = pl.kernel(
        _agg_body,
        out_type=jax.ShapeDtypeStruct((NC, N_PAD, HID), jnp.float32),
        mesh=_mesh,
        scratch_types=[
            pltpu.VMEM((EPT,), jnp.int32),
            pltpu.VMEM((EPT,), jnp.int32),
            pltpu.VMEM((EPT, HID), jnp.float32),
            pltpu.VMEM((ZCHUNK, HID), jnp.float32),
            pltpu.VMEM_SHARED((N_PAD, HID), jnp.float32),
            pltpu.SemaphoreType.DMA,
            pltpu.SemaphoreType.DMA,
        ],
        compiler_params=_sc_params,
    )
    return k(h, ei)


_BM = 1024
_BMP = _BM // (128 // HID)
PROWS = N_PAD // (128 // HID)


def _packmask():
    r8 = 128 // HID
    m_row = lax.broadcasted_iota(jnp.int32, (_BM, 128), 0)
    c_col = lax.broadcasted_iota(jnp.int32, (_BM, 128), 1)
    return (m_row % r8 == c_col // HID).astype(jnp.float32)


def _grpsum(y):
    return jnp.sum(y.reshape(_BMP, 128 // HID, 128), axis=1)


def _mm1_body(x_ref, w1e_ref, degp_ref, hp_ref, nrmd_ref, nrms_ref):
    degp = degp_ref[...]
    deg_s = jnp.maximum(jnp.sum(degp[:NW], axis=0), 1.0)
    deg_d = jnp.maximum(jnp.sum(degp[NW:], axis=0), 1.0)
    mask = _packmask()
    nrms_col = lax.rsqrt(deg_s)[:, None]
    nrmd_col = lax.rsqrt(deg_d)[:, None]
    nrms_ref[...] = _grpsum(nrms_col * mask)
    nrmd_ref[...] = _grpsum(nrmd_col * mask)
    mm = jnp.dot(x_ref[...], w1e_ref[...], preferred_element_type=jnp.float32)
    hp_ref[...] = _grpsum(mm * (nrms_col * mask))


def _tc_mm1(features, W1e, degp):
    grid = N_PAD // _BM
    return pl.pallas_call(
        _mm1_body,
        grid=(grid,),
        in_specs=[
            pl.BlockSpec((_BM, F_IN), lambda i: (i, 0)),
            pl.BlockSpec((F_IN, 128), lambda i: (0, 0)),
            pl.BlockSpec((2 * NW, _BM), lambda i: (0, i)),
        ],
        out_specs=[
            pl.BlockSpec((_BMP, 128), lambda i: (i, 0)),
            pl.BlockSpec((_BMP, 128), lambda i: (i, 0)),
            pl.BlockSpec((_BMP, 128), lambda i: (i, 0)),
        ],
        out_shape=[
            jax.ShapeDtypeStruct((PROWS, 128), jnp.float32),
            jax.ShapeDtypeStruct((PROWS, 128), jnp.float32),
            jax.ShapeDtypeStruct((PROWS, 128), jnp.float32),
        ],
    )(features, W1e, degp)


def _mid_body(p_ref, nrmd_ref, nrms_ref, b1_ref, w2bd_ref, h2p_ref):
    agg = p_ref[0] + p_ref[1]
    t = jnp.maximum(agg * nrmd_ref[...] + b1_ref[...], 0.0)
    mm = jnp.dot(t, w2bd_ref[...], preferred_element_type=jnp.float32)
    h2p_ref[...] = mm * nrms_ref[...]


def _tc_mid(p1, nrmd_p, nrms_p, b1t, W2bd):
    grid = PROWS // _BMP
    return pl.pallas_call(
        _mid_body,
        grid=(grid,),
        in_specs=[
            pl.BlockSpec((NC, _BMP, 128), lambda i: (0, i, 0)),
            pl.BlockSpec((_BMP, 128), lambda i: (i, 0)),
            pl.BlockSpec((_BMP, 128), lambda i: (i, 0)),
            pl.BlockSpec((1, 128), lambda i: (0, 0)),
            pl.BlockSpec((128, 128), lambda i: (0, 0)),
        ],
        out_specs=pl.BlockSpec((_BMP, 128), lambda i: (i, 0)),
        out_shape=jax.ShapeDtypeStruct((PROWS, 128), jnp.float32),
    )(p1, nrmd_p, nrms_p, b1t, W2bd)


def _final_body(p_ref, nrmd_ref, b2_ref, out_ref):
    agg = p_ref[0] + p_ref[1]
    out_ref[...] = agg * nrmd_ref[...] + b2_ref[...]


def _tc_final(p2, nrmd_p, b2t):
    grid = PROWS // _BMP
    return pl.pallas_call(
        _final_body,
        grid=(grid,),
        in_specs=[
            pl.BlockSpec((NC, _BMP, 128), lambda i: (0, i, 0)),
            pl.BlockSpec((_BMP, 128), lambda i: (i, 0)),
            pl.BlockSpec((1, 128), lambda i: (0, 0)),
        ],
        out_specs=pl.BlockSpec((_BMP, 128), lambda i: (i, 0)),
        out_shape=jax.ShapeDtypeStruct((PROWS, 128), jnp.float32),
    )(p2, nrmd_p, b2t)


def kernel(features, edge_index, W1, b1, W2, b2):
    ei = edge_index.astype(jnp.int32)
    rep = 128 // HID
    b1t = jnp.tile(b1, rep).reshape(1, 128)
    b2t = jnp.tile(b2, rep).reshape(1, 128)
    W1e = jnp.tile(W1, (1, rep))
    W2bd = jnp.kron(jnp.eye(rep, dtype=jnp.float32), W2)

    degp = _sc_degrees(ei)
    h1p, nrmd_p, nrms_p = _tc_mm1(features, W1e, degp)
    p1 = _sc_agg(h1p.reshape(N_PAD, HID), ei)
    h2p = _tc_mid(p1.reshape(NC, PROWS, 128), nrmd_p, nrms_p, b1t, W2bd)
    p2 = _sc_agg(h2p.reshape(N_PAD, HID), ei)
    outp = _tc_final(p2.reshape(NC, PROWS, 128), nrmd_p, b2t)
    return outp.reshape(N_PAD, HID)[:N_NODES]
